# Initial kernel scaffold; baseline (speedup 1.0000x reference)
import jax
import jax.numpy as jnp
from jax.experimental import pallas as pl


def kernel(logits, output_tokens, presence_penalties, frequency_penalties, temperatures, top_ps, top_ks):
    raise NotImplementedError("write your pallas kernel here")


# speedup vs baseline: 20.4657x; 20.4657x over previous
import functools

import jax
import jax.numpy as jnp
from jax import lax
from jax.experimental import pallas as pl
from jax.experimental.pallas import tpu as pltpu
from jax.experimental.pallas import tpu_sc as plsc

_PADH = 256
_NW = 32
_ONE_KEY = 0x3F800001


def _prep_body(tok_ref, fp_ref, pp_ref, tokp_ref, s1_ref, s2_ref, fst_ref):
    rb, h = tok_ref.shape
    t = tok_ref[...]
    fp = fp_ref[...]
    pp = pp_ref[...]
    eq = t[:, :, None] == t[:, None, :]
    cnt = jnp.sum(eq.astype(jnp.float32), axis=1)
    ik = lax.broadcasted_iota(jnp.int32, (rb, h, h), 1)
    ij = lax.broadcasted_iota(jnp.int32, (rb, h, h), 2)
    prior = jnp.sum((eq & (ik < ij)).astype(jnp.int32), axis=1)
    first = (prior == 0).astype(jnp.int32)
    pad = _PADH - h
    zi = jnp.zeros((rb, pad), jnp.int32)
    zf = jnp.zeros((rb, pad), jnp.float32)
    tokp_ref[...] = jnp.concatenate([t, zi], axis=1)
    s1_ref[...] = jnp.concatenate([fp * cnt, zf], axis=1)
    s2_ref[...] = jnp.concatenate([jnp.broadcast_to(pp, (rb, h)), zf], axis=1)
    fst_ref[...] = jnp.concatenate([first, zi], axis=1)


def _sc_penalize_body(tok_hbm, s1_hbm, s2_hbm, fst_hbm, logits_hbm,
                      out_hbm, row_v, tok_v, s1_v, s2_v, fst_v):
    rows_per_worker = logits_hbm.shape[0] // _NW
    wid = lax.axis_index("s") * 2 + lax.axis_index("c")
    for r in range(rows_per_worker):
        b = wid * rows_per_worker + r
        pltpu.sync_copy(tok_hbm.at[b], tok_v)
        pltpu.sync_copy(s1_hbm.at[b], s1_v)
        pltpu.sync_copy(s2_hbm.at[b], s2_v)
        pltpu.sync_copy(logits_hbm.at[b], row_v)
        pltpu.sync_copy(fst_hbm.at[b], fst_v)
        for c in range(_PADH // 16):
            sl = pl.ds(c * 16, 16)
            idx = tok_v[sl]
            vals = plsc.load_gather(row_v, [idx])
            newv = (vals - s1_v[sl]) - s2_v[sl]
            plsc.store_scatter(row_v, [idx], newv, mask=fst_v[sl] != 0)
        pltpu.sync_copy(row_v, out_hbm.at[b])


def _main_body(x_ref, t_ref, tp_ref, tk_ref, out_ref, tok_ref):
    rb, v = x_ref.shape
    lp = x_ref[...] / t_ref[...]
    m = jnp.max(lp, axis=1, keepdims=True)
    e = jnp.exp(lp - m)
    z = jnp.sum(e, axis=1, keepdims=True)
    p = e / z
    iota = lax.broadcasted_iota(jnp.int32, (rb, v), 1)
    amax = jnp.max(p, axis=1, keepdims=True)
    tok_ref[...] = jnp.min(jnp.where(p == amax, iota, v), axis=1, keepdims=True)
    kp = lax.bitcast_convert_type(p, jnp.int32)
    topk = tk_ref[...]
    topp = tp_ref[...]

    def body(_, carry):
        lo, hi = carry
        mid = (lo + hi) >> 1
        msk = kp > mid
        cnt = jnp.sum(msk.astype(jnp.int32), axis=1, keepdims=True)
        sp = jnp.sum(jnp.where(msk, p, 0.0), axis=1, keepdims=True)
        ok = (cnt < topk) & (sp <= topp)
        return jnp.where(ok, lo, mid), jnp.where(ok, mid, hi)

    lo0 = jnp.full((rb, 1), -1, jnp.int32)
    hi0 = jnp.full((rb, 1), _ONE_KEY, jnp.int32)
    lo, _ = lax.fori_loop(0, 32, body, (lo0, hi0))
    kept = kp > lo
    s = jnp.sum(jnp.where(kept, p, 0.0), axis=1, keepdims=True)
    out_ref[...] = jnp.where(kept, p / s, 0.0)


def kernel(logits, output_tokens, presence_penalties, frequency_penalties,
           temperatures, top_ps, top_ks):
    b, v = logits.shape
    h = output_tokens.shape[1]
    rb = 8

    tok_pad, s1, s2, fst = pl.pallas_call(
        _prep_body,
        grid=(b // rb,),
        in_specs=[
            pl.BlockSpec((rb, h), lambda i: (i, 0)),
            pl.BlockSpec((rb, 1), lambda i: (i, 0)),
            pl.BlockSpec((rb, 1), lambda i: (i, 0)),
        ],
        out_specs=[pl.BlockSpec((rb, _PADH), lambda i: (i, 0))] * 4,
        out_shape=[
            jax.ShapeDtypeStruct((b, _PADH), jnp.int32),
            jax.ShapeDtypeStruct((b, _PADH), jnp.float32),
            jax.ShapeDtypeStruct((b, _PADH), jnp.float32),
            jax.ShapeDtypeStruct((b, _PADH), jnp.int32),
        ],
    )(output_tokens,
      frequency_penalties.reshape(b, 1),
      presence_penalties.reshape(b, 1))

    sc_pen = functools.partial(
        pl.kernel,
        mesh=plsc.VectorSubcoreMesh(core_axis_name="c", subcore_axis_name="s"),
        compiler_params=pltpu.CompilerParams(needs_layout_passes=False),
        out_type=jax.ShapeDtypeStruct((b, v), jnp.float32),
        scratch_types=[
            pltpu.VMEM((v,), jnp.float32),
            pltpu.VMEM((_PADH,), jnp.int32),
            pltpu.VMEM((_PADH,), jnp.float32),
            pltpu.VMEM((_PADH,), jnp.float32),
            pltpu.VMEM((_PADH,), jnp.int32),
        ],
    )(_sc_penalize_body)
    lpen = sc_pen(tok_pad, s1, s2, fst, logits)

    probs, tok = pl.pallas_call(
        _main_body,
        grid=(b // rb,),
        in_specs=[
            pl.BlockSpec((rb, v), lambda i: (i, 0)),
            pl.BlockSpec((rb, 1), lambda i: (i, 0)),
            pl.BlockSpec((rb, 1), lambda i: (i, 0)),
            pl.BlockSpec((rb, 1), lambda i: (i, 0)),
        ],
        out_specs=[
            pl.BlockSpec((rb, v), lambda i: (i, 0)),
            pl.BlockSpec((rb, 1), lambda i: (i, 0)),
        ],
        out_shape=[
            jax.ShapeDtypeStruct((b, v), jnp.float32),
            jax.ShapeDtypeStruct((b, 1), jnp.int32),
        ],
    )(lpen,
      temperatures.reshape(b, 1),
      top_ps.reshape(b, 1),
      top_ks.reshape(b, 1))

    return probs, tok.reshape(b)


# speedup vs baseline: 21.3896x; 1.0451x over previous
import functools

import jax
import jax.numpy as jnp
from jax import lax
from jax.experimental import pallas as pl
from jax.experimental.pallas import tpu as pltpu
from jax.experimental.pallas import tpu_sc as plsc

_PADH = 256
_NW = 32
_ONE_KEY = 0x3F800001


def _prep_body(tok_ref, fp_ref, pp_ref, tokp_ref, s1_ref, s2_ref, fst_ref):
    rb, h = tok_ref.shape
    t = tok_ref[...]
    fp = fp_ref[...]
    pp = pp_ref[...]
    eq = t[:, :, None] == t[:, None, :]
    cnt = jnp.sum(eq.astype(jnp.float32), axis=1)
    ik = lax.broadcasted_iota(jnp.int32, (rb, h, h), 1)
    ij = lax.broadcasted_iota(jnp.int32, (rb, h, h), 2)
    prior = jnp.sum((eq & (ik < ij)).astype(jnp.int32), axis=1)
    first = (prior == 0).astype(jnp.int32)
    pad = _PADH - h
    zi = jnp.zeros((rb, pad), jnp.int32)
    zf = jnp.zeros((rb, pad), jnp.float32)
    tokp_ref[...] = jnp.concatenate([t, zi], axis=1)
    s1_ref[...] = jnp.concatenate([fp * cnt, zf], axis=1)
    s2_ref[...] = jnp.concatenate([jnp.broadcast_to(pp, (rb, h)), zf], axis=1)
    fst_ref[...] = jnp.concatenate([first, zi], axis=1)


def _sc_penalize_body(tok_hbm, s1_hbm, s2_hbm, fst_hbm, logits_hbm,
                      out_hbm, row_v, tok_v, s1_v, s2_v, fst_v):
    rows_per_worker = logits_hbm.shape[0] // _NW
    wid = lax.axis_index("s") * 2 + lax.axis_index("c")
    for r in range(rows_per_worker):
        b = wid * rows_per_worker + r
        pltpu.sync_copy(tok_hbm.at[b], tok_v)
        pltpu.sync_copy(s1_hbm.at[b], s1_v)
        pltpu.sync_copy(s2_hbm.at[b], s2_v)
        pltpu.sync_copy(logits_hbm.at[b], row_v)
        pltpu.sync_copy(fst_hbm.at[b], fst_v)
        for c in range(_PADH // 16):
            sl = pl.ds(c * 16, 16)
            idx = tok_v[sl]
            vals = plsc.load_gather(row_v, [idx])
            newv = (vals - s1_v[sl]) - s2_v[sl]
            plsc.store_scatter(row_v, [idx], newv, mask=fst_v[sl] != 0)
        pltpu.sync_copy(row_v, out_hbm.at[b])


def _main_body(x_ref, t_ref, tp_ref, tk_ref, out_ref, tok_ref):
    rb, v = x_ref.shape
    lp = x_ref[...] / t_ref[...]
    m = jnp.max(lp, axis=1, keepdims=True)
    e = jnp.exp(lp - m)
    z = jnp.sum(e, axis=1, keepdims=True)
    p = e / z
    iota = lax.broadcasted_iota(jnp.int32, (rb, v), 1)
    amax = jnp.max(p, axis=1, keepdims=True)
    tok_ref[...] = jnp.min(jnp.where(p == amax, iota, v), axis=1, keepdims=True)
    topk = tk_ref[...]
    topp = tp_ref[...]

    def body(_, carry):
        lo, hi = carry
        mid = (lo + hi) >> 1
        midf = lax.bitcast_convert_type(mid, jnp.float32)
        msk = p > midf
        cnt = jnp.sum(msk.astype(jnp.int32), axis=1, keepdims=True)
        sp = jnp.sum(jnp.where(msk, p, 0.0), axis=1, keepdims=True)
        ok = (cnt < topk) & (sp <= topp)
        return jnp.where(ok, lo, mid), jnp.where(ok, mid, hi)

    lo0 = jnp.full((rb, 1), -1, jnp.int32)
    hi0 = jnp.full((rb, 1), _ONE_KEY, jnp.int32)
    _, hi = lax.fori_loop(0, 30, body, (lo0, hi0))
    kept = p >= lax.bitcast_convert_type(hi, jnp.float32)
    s = jnp.sum(jnp.where(kept, p, 0.0), axis=1, keepdims=True)
    out_ref[...] = jnp.where(kept, p / s, 0.0)


def kernel(logits, output_tokens, presence_penalties, frequency_penalties,
           temperatures, top_ps, top_ks):
    b, v = logits.shape
    h = output_tokens.shape[1]
    rb = 8

    tok_pad, s1, s2, fst = pl.pallas_call(
        _prep_body,
        grid=(b // rb,),
        in_specs=[
            pl.BlockSpec((rb, h), lambda i: (i, 0)),
            pl.BlockSpec((rb, 1), lambda i: (i, 0)),
            pl.BlockSpec((rb, 1), lambda i: (i, 0)),
        ],
        out_specs=[pl.BlockSpec((rb, _PADH), lambda i: (i, 0))] * 4,
        out_shape=[
            jax.ShapeDtypeStruct((b, _PADH), jnp.int32),
            jax.ShapeDtypeStruct((b, _PADH), jnp.float32),
            jax.ShapeDtypeStruct((b, _PADH), jnp.float32),
            jax.ShapeDtypeStruct((b, _PADH), jnp.int32),
        ],
    )(output_tokens,
      frequency_penalties.reshape(b, 1),
      presence_penalties.reshape(b, 1))

    sc_pen = functools.partial(
        pl.kernel,
        mesh=plsc.VectorSubcoreMesh(core_axis_name="c", subcore_axis_name="s"),
        compiler_params=pltpu.CompilerParams(needs_layout_passes=False),
        out_type=jax.ShapeDtypeStruct((b, v), jnp.float32),
        scratch_types=[
            pltpu.VMEM((v,), jnp.float32),
            pltpu.VMEM((_PADH,), jnp.int32),
            pltpu.VMEM((_PADH,), jnp.float32),
            pltpu.VMEM((_PADH,), jnp.float32),
            pltpu.VMEM((_PADH,), jnp.int32),
        ],
    )(_sc_penalize_body)
    lpen = sc_pen(tok_pad, s1, s2, fst, logits)

    probs, tok = pl.pallas_call(
        _main_body,
        grid=(b // rb,),
        in_specs=[
            pl.BlockSpec((rb, v), lambda i: (i, 0)),
            pl.BlockSpec((rb, 1), lambda i: (i, 0)),
            pl.BlockSpec((rb, 1), lambda i: (i, 0)),
            pl.BlockSpec((rb, 1), lambda i: (i, 0)),
        ],
        out_specs=[
            pl.BlockSpec((rb, v), lambda i: (i, 0)),
            pl.BlockSpec((rb, 1), lambda i: (i, 0)),
        ],
        out_shape=[
            jax.ShapeDtypeStruct((b, v), jnp.float32),
            jax.ShapeDtypeStruct((b, 1), jnp.int32),
        ],
    )(lpen,
      temperatures.reshape(b, 1),
      top_ps.reshape(b, 1),
      top_ks.reshape(b, 1))

    return probs, tok.reshape(b)


# speedup vs baseline: 28.7118x; 1.3423x over previous
import functools

import jax
import jax.numpy as jnp
from jax import lax
from jax.experimental import pallas as pl
from jax.experimental.pallas import tpu as pltpu
from jax.experimental.pallas import tpu_sc as plsc

_PADH = 256
_NW = 32
_ONE_KEY = 0x3F800001


def _prep_body(tok_ref, fp_ref, pp_ref, tokp_ref, s1_ref, s2_ref, fst_ref):
    rb, h = tok_ref.shape
    t = tok_ref[...]
    fp = fp_ref[...]
    pp = pp_ref[...]
    eq = t[:, :, None] == t[:, None, :]
    cnt = jnp.sum(eq.astype(jnp.float32), axis=1)
    ik = lax.broadcasted_iota(jnp.int32, (rb, h, h), 1)
    ij = lax.broadcasted_iota(jnp.int32, (rb, h, h), 2)
    prior = jnp.sum((eq & (ik < ij)).astype(jnp.int32), axis=1)
    first = (prior == 0).astype(jnp.int32)
    pad = _PADH - h
    zi = jnp.zeros((rb, pad), jnp.int32)
    zf = jnp.zeros((rb, pad), jnp.float32)
    tokp_ref[...] = jnp.concatenate([t, zi], axis=1)
    s1_ref[...] = jnp.concatenate([fp * cnt, zf], axis=1)
    s2_ref[...] = jnp.concatenate([jnp.broadcast_to(pp, (rb, h)), zf], axis=1)
    fst_ref[...] = jnp.concatenate([first, zi], axis=1)


def _sc_penalize_body(tok_hbm, s1_hbm, s2_hbm, fst_hbm, logits_hbm,
                      out_hbm, row_v, tok_v, s1_v, s2_v, fst_v):
    rows_per_worker = logits_hbm.shape[0] // _NW
    wid = lax.axis_index("s") * 2 + lax.axis_index("c")
    for r in range(rows_per_worker):
        b = wid * rows_per_worker + r
        pltpu.sync_copy(tok_hbm.at[b], tok_v)
        pltpu.sync_copy(s1_hbm.at[b], s1_v)
        pltpu.sync_copy(s2_hbm.at[b], s2_v)
        pltpu.sync_copy(logits_hbm.at[b], row_v)
        pltpu.sync_copy(fst_hbm.at[b], fst_v)
        for c in range(_PADH // 16):
            sl = pl.ds(c * 16, 16)
            idx = tok_v[sl]
            vals = plsc.load_gather(row_v, [idx])
            newv = (vals - s1_v[sl]) - s2_v[sl]
            plsc.store_scatter(row_v, [idx], newv, mask=fst_v[sl] != 0)
        pltpu.sync_copy(row_v, out_hbm.at[b])


def _main_body(x_ref, t_ref, tp_ref, tk_ref, out_ref, tok_ref):
    rb, v = x_ref.shape
    nck = 8
    step = ((v // nck) // 128) * 128
    bounds = [(k * step, (k + 1) * step if k < nck - 1 else v)
              for k in range(nck)]

    def rsum(x):
        return functools.reduce(jnp.add, [
            jnp.sum(x[:, s:e2], axis=1, keepdims=True) for s, e2 in bounds])

    def rmax(x):
        return functools.reduce(jnp.maximum, [
            jnp.max(x[:, s:e2], axis=1, keepdims=True) for s, e2 in bounds])

    def rmin(x):
        return functools.reduce(jnp.minimum, [
            jnp.min(x[:, s:e2], axis=1, keepdims=True) for s, e2 in bounds])

    lp = x_ref[...] / t_ref[...]
    m = rmax(lp)
    e = jnp.exp(lp - m)
    z = rsum(e)
    p = e / z
    iota = lax.broadcasted_iota(jnp.int32, (rb, v), 1)
    amax = rmax(p)
    tok_ref[...] = rmin(jnp.where(p == amax, iota, v))
    topk = tk_ref[...]
    topp = tp_ref[...]

    def body(_, carry):
        lo, hi = carry
        mid = (lo + hi) >> 1
        midf = lax.bitcast_convert_type(mid, jnp.float32)
        cs, ss = [], []
        for s, e2 in bounds:
            pc = p[:, s:e2]
            mc = pc > midf
            cs.append(jnp.sum(mc.astype(jnp.int32), axis=1, keepdims=True))
            ss.append(jnp.sum(jnp.where(mc, pc, 0.0), axis=1, keepdims=True))
        cnt = functools.reduce(jnp.add, cs)
        sp = functools.reduce(jnp.add, ss)
        ok = (cnt < topk) & (sp <= topp)
        return jnp.where(ok, lo, mid), jnp.where(ok, mid, hi)

    lo0 = jnp.full((rb, 1), -1, jnp.int32)
    hi0 = jnp.full((rb, 1), _ONE_KEY, jnp.int32)
    _, hi = lax.fori_loop(0, 30, body, (lo0, hi0))
    kept = p >= lax.bitcast_convert_type(hi, jnp.float32)
    s = rsum(jnp.where(kept, p, 0.0))
    out_ref[...] = jnp.where(kept, p / s, 0.0)


def kernel(logits, output_tokens, presence_penalties, frequency_penalties,
           temperatures, top_ps, top_ks):
    b, v = logits.shape
    h = output_tokens.shape[1]
    rb = 8

    tok_pad, s1, s2, fst = pl.pallas_call(
        _prep_body,
        grid=(b // rb,),
        in_specs=[
            pl.BlockSpec((rb, h), lambda i: (i, 0)),
            pl.BlockSpec((rb, 1), lambda i: (i, 0)),
            pl.BlockSpec((rb, 1), lambda i: (i, 0)),
        ],
        out_specs=[pl.BlockSpec((rb, _PADH), lambda i: (i, 0))] * 4,
        out_shape=[
            jax.ShapeDtypeStruct((b, _PADH), jnp.int32),
            jax.ShapeDtypeStruct((b, _PADH), jnp.float32),
            jax.ShapeDtypeStruct((b, _PADH), jnp.float32),
            jax.ShapeDtypeStruct((b, _PADH), jnp.int32),
        ],
    )(output_tokens,
      frequency_penalties.reshape(b, 1),
      presence_penalties.reshape(b, 1))

    sc_pen = functools.partial(
        pl.kernel,
        mesh=plsc.VectorSubcoreMesh(core_axis_name="c", subcore_axis_name="s"),
        compiler_params=pltpu.CompilerParams(needs_layout_passes=False),
        out_type=jax.ShapeDtypeStruct((b, v), jnp.float32),
        scratch_types=[
            pltpu.VMEM((v,), jnp.float32),
            pltpu.VMEM((_PADH,), jnp.int32),
            pltpu.VMEM((_PADH,), jnp.float32),
            pltpu.VMEM((_PADH,), jnp.float32),
            pltpu.VMEM((_PADH,), jnp.int32),
        ],
    )(_sc_penalize_body)
    lpen = sc_pen(tok_pad, s1, s2, fst, logits)

    probs, tok = pl.pallas_call(
        _main_body,
        grid=(b // rb,),
        in_specs=[
            pl.BlockSpec((rb, v), lambda i: (i, 0)),
            pl.BlockSpec((rb, 1), lambda i: (i, 0)),
            pl.BlockSpec((rb, 1), lambda i: (i, 0)),
            pl.BlockSpec((rb, 1), lambda i: (i, 0)),
        ],
        out_specs=[
            pl.BlockSpec((rb, v), lambda i: (i, 0)),
            pl.BlockSpec((rb, 1), lambda i: (i, 0)),
        ],
        out_shape=[
            jax.ShapeDtypeStruct((b, v), jnp.float32),
            jax.ShapeDtypeStruct((b, 1), jnp.int32),
        ],
    )(lpen,
      temperatures.reshape(b, 1),
      top_ps.reshape(b, 1),
      top_ks.reshape(b, 1))

    return probs, tok.reshape(b)


# speedup vs baseline: 29.6511x; 1.0327x over previous
import functools

import jax
import jax.numpy as jnp
from jax import lax
from jax.experimental import pallas as pl
from jax.experimental.pallas import tpu as pltpu
from jax.experimental.pallas import tpu_sc as plsc

_PADH = 256
_NW = 32
_ONE_KEY = 0x3F800001


def _prep_body(tok_ref, fp_ref, pp_ref, tokp_ref, s1_ref, s2_ref, fst_ref):
    rb, h = tok_ref.shape
    t = tok_ref[...]
    fp = fp_ref[...]
    pp = pp_ref[...]
    eq = t[:, :, None] == t[:, None, :]
    cnt = jnp.sum(eq.astype(jnp.float32), axis=1)
    ik = lax.broadcasted_iota(jnp.int32, (rb, h, h), 1)
    ij = lax.broadcasted_iota(jnp.int32, (rb, h, h), 2)
    prior = jnp.sum((eq & (ik < ij)).astype(jnp.int32), axis=1)
    first = (prior == 0).astype(jnp.int32)
    pad = _PADH - h
    zi = jnp.zeros((rb, pad), jnp.int32)
    zf = jnp.zeros((rb, pad), jnp.float32)
    tokp_ref[...] = jnp.concatenate([t, zi], axis=1)
    s1_ref[...] = jnp.concatenate([fp * cnt, zf], axis=1)
    s2_ref[...] = jnp.concatenate([jnp.broadcast_to(pp, (rb, h)), zf], axis=1)
    fst_ref[...] = jnp.concatenate([first, zi], axis=1)


def _sc_penalize_body(tok_hbm, s1_hbm, s2_hbm, fst_hbm, logits_hbm,
                      out_hbm, row_v, tok_v, s1_v, s2_v, fst_v):
    rows_per_worker = logits_hbm.shape[0] // _NW
    wid = lax.axis_index("s") * 2 + lax.axis_index("c")
    for r in range(rows_per_worker):
        b = wid * rows_per_worker + r
        pltpu.sync_copy(tok_hbm.at[b], tok_v)
        pltpu.sync_copy(s1_hbm.at[b], s1_v)
        pltpu.sync_copy(s2_hbm.at[b], s2_v)
        pltpu.sync_copy(logits_hbm.at[b], row_v)
        pltpu.sync_copy(fst_hbm.at[b], fst_v)
        for c in range(_PADH // 16):
            sl = pl.ds(c * 16, 16)
            idx = tok_v[sl]
            vals = plsc.load_gather(row_v, [idx])
            newv = (vals - s1_v[sl]) - s2_v[sl]
            plsc.store_scatter(row_v, [idx], newv, mask=fst_v[sl] != 0)
        pltpu.sync_copy(row_v, out_hbm.at[b])


def _main_body(x_ref, t_ref, tp_ref, tk_ref, out_ref, tok_ref):
    rb, v = x_ref.shape
    nck = 16
    step = ((v // nck) // 128) * 128
    bounds = [(k * step, (k + 1) * step if k < nck - 1 else v)
              for k in range(nck)]

    def rsum(x):
        return functools.reduce(jnp.add, [
            jnp.sum(x[:, s:e2], axis=1, keepdims=True) for s, e2 in bounds])

    def rmax(x):
        return functools.reduce(jnp.maximum, [
            jnp.max(x[:, s:e2], axis=1, keepdims=True) for s, e2 in bounds])

    def rmin(x):
        return functools.reduce(jnp.minimum, [
            jnp.min(x[:, s:e2], axis=1, keepdims=True) for s, e2 in bounds])

    lp = x_ref[...] / t_ref[...]
    m = rmax(lp)
    e = jnp.exp(lp - m)
    z = rsum(e)
    p = e / z
    iota = lax.broadcasted_iota(jnp.int32, (rb, v), 1)
    amax = rmax(p)
    tok_ref[...] = rmin(jnp.where(p == amax, iota, v))
    topk = tk_ref[...]
    topp = tp_ref[...]

    def body(_, carry):
        lo, hi = carry
        mid = (lo + hi) >> 1
        midf = lax.bitcast_convert_type(mid, jnp.float32)
        cs, ss = [], []
        for s, e2 in bounds:
            pc = p[:, s:e2]
            mc = pc > midf
            cs.append(jnp.sum(mc.astype(jnp.int32), axis=1, keepdims=True))
            ss.append(jnp.sum(jnp.where(mc, pc, 0.0), axis=1, keepdims=True))
        cnt = functools.reduce(jnp.add, cs)
        sp = functools.reduce(jnp.add, ss)
        ok = (cnt < topk) & (sp <= topp)
        return jnp.where(ok, lo, mid), jnp.where(ok, mid, hi)

    lo0 = jnp.full((rb, 1), -1, jnp.int32)
    hi0 = jnp.full((rb, 1), _ONE_KEY, jnp.int32)
    _, hi = lax.fori_loop(0, 30, body, (lo0, hi0))
    kept = p >= lax.bitcast_convert_type(hi, jnp.float32)
    s = rsum(jnp.where(kept, p, 0.0))
    out_ref[...] = jnp.where(kept, p / s, 0.0)


def kernel(logits, output_tokens, presence_penalties, frequency_penalties,
           temperatures, top_ps, top_ks):
    b, v = logits.shape
    h = output_tokens.shape[1]
    rb = 8

    tok_pad, s1, s2, fst = pl.pallas_call(
        _prep_body,
        grid=(b // rb,),
        in_specs=[
            pl.BlockSpec((rb, h), lambda i: (i, 0)),
            pl.BlockSpec((rb, 1), lambda i: (i, 0)),
            pl.BlockSpec((rb, 1), lambda i: (i, 0)),
        ],
        out_specs=[pl.BlockSpec((rb, _PADH), lambda i: (i, 0))] * 4,
        out_shape=[
            jax.ShapeDtypeStruct((b, _PADH), jnp.int32),
            jax.ShapeDtypeStruct((b, _PADH), jnp.float32),
            jax.ShapeDtypeStruct((b, _PADH), jnp.float32),
            jax.ShapeDtypeStruct((b, _PADH), jnp.int32),
        ],
    )(output_tokens,
      frequency_penalties.reshape(b, 1),
      presence_penalties.reshape(b, 1))

    sc_pen = functools.partial(
        pl.kernel,
        mesh=plsc.VectorSubcoreMesh(core_axis_name="c", subcore_axis_name="s"),
        compiler_params=pltpu.CompilerParams(needs_layout_passes=False),
        out_type=jax.ShapeDtypeStruct((b, v), jnp.float32),
        scratch_types=[
            pltpu.VMEM((v,), jnp.float32),
            pltpu.VMEM((_PADH,), jnp.int32),
            pltpu.VMEM((_PADH,), jnp.float32),
            pltpu.VMEM((_PADH,), jnp.float32),
            pltpu.VMEM((_PADH,), jnp.int32),
        ],
    )(_sc_penalize_body)
    lpen = sc_pen(tok_pad, s1, s2, fst, logits)

    probs, tok = pl.pallas_call(
        _main_body,
        grid=(b // rb,),
        in_specs=[
            pl.BlockSpec((rb, v), lambda i: (i, 0)),
            pl.BlockSpec((rb, 1), lambda i: (i, 0)),
            pl.BlockSpec((rb, 1), lambda i: (i, 0)),
            pl.BlockSpec((rb, 1), lambda i: (i, 0)),
        ],
        out_specs=[
            pl.BlockSpec((rb, v), lambda i: (i, 0)),
            pl.BlockSpec((rb, 1), lambda i: (i, 0)),
        ],
        out_shape=[
            jax.ShapeDtypeStruct((b, v), jnp.float32),
            jax.ShapeDtypeStruct((b, 1), jnp.int32),
        ],
    )(lpen,
      temperatures.reshape(b, 1),
      top_ps.reshape(b, 1),
      top_ks.reshape(b, 1))

    return probs, tok.reshape(b)


# speedup vs baseline: 32.8392x; 1.1075x over previous
import functools

import jax
import jax.numpy as jnp
from jax import lax
from jax.experimental import pallas as pl
from jax.experimental.pallas import tpu as pltpu
from jax.experimental.pallas import tpu_sc as plsc

_PADH = 256
_NW = 32
_ONE_KEY = 0x3F800001


def _prep_body(tok_ref, fp_ref, pp_ref, tokp_ref, s1_ref, s2_ref, fst_ref):
    rb, h = tok_ref.shape
    t = tok_ref[...]
    fp = fp_ref[...]
    pp = pp_ref[...]
    eq = t[:, :, None] == t[:, None, :]
    cnt = jnp.sum(eq.astype(jnp.float32), axis=1)
    ik = lax.broadcasted_iota(jnp.int32, (rb, h, h), 1)
    ij = lax.broadcasted_iota(jnp.int32, (rb, h, h), 2)
    prior = jnp.sum((eq & (ik < ij)).astype(jnp.int32), axis=1)
    first = (prior == 0).astype(jnp.int32)
    pad = _PADH - h
    zi = jnp.zeros((rb, pad), jnp.int32)
    zf = jnp.zeros((rb, pad), jnp.float32)
    tokp_ref[...] = jnp.concatenate([t, zi], axis=1)
    s1_ref[...] = jnp.concatenate([fp * cnt, zf], axis=1)
    s2_ref[...] = jnp.concatenate([jnp.broadcast_to(pp, (rb, h)), zf], axis=1)
    fst_ref[...] = jnp.concatenate([first, zi], axis=1)


def _sc_penalize_body(tok_hbm, s1_hbm, s2_hbm, fst_hbm, logits_hbm,
                      out_hbm, row_v, tok_v, s1_v, s2_v, fst_v):
    rows_per_worker = logits_hbm.shape[0] // _NW
    wid = lax.axis_index("s") * 2 + lax.axis_index("c")
    for r in range(rows_per_worker):
        b = wid * rows_per_worker + r
        pltpu.sync_copy(tok_hbm.at[b], tok_v)
        pltpu.sync_copy(s1_hbm.at[b], s1_v)
        pltpu.sync_copy(s2_hbm.at[b], s2_v)
        pltpu.sync_copy(logits_hbm.at[b], row_v)
        pltpu.sync_copy(fst_hbm.at[b], fst_v)
        for c in range(_PADH // 16):
            sl = pl.ds(c * 16, 16)
            idx = tok_v[sl]
            vals = plsc.load_gather(row_v, [idx])
            newv = (vals - s1_v[sl]) - s2_v[sl]
            plsc.store_scatter(row_v, [idx], newv, mask=fst_v[sl] != 0)
        pltpu.sync_copy(row_v, out_hbm.at[b])


def _main_body(x_ref, t_ref, tp_ref, tk_ref, out_ref, tok_ref):
    rb, v = x_ref.shape
    nck = 16
    step = ((v // nck) // 128) * 128
    bounds = [(k * step, (k + 1) * step if k < nck - 1 else v)
              for k in range(nck)]

    def rsum(x):
        return functools.reduce(jnp.add, [
            jnp.sum(x[:, s:e2], axis=1, keepdims=True) for s, e2 in bounds])

    def rmax(x):
        return functools.reduce(jnp.maximum, [
            jnp.max(x[:, s:e2], axis=1, keepdims=True) for s, e2 in bounds])

    def rmin(x):
        return functools.reduce(jnp.minimum, [
            jnp.min(x[:, s:e2], axis=1, keepdims=True) for s, e2 in bounds])

    lp = x_ref[...] / t_ref[...]
    m = rmax(lp)
    e = jnp.exp(lp - m)
    z = rsum(e)
    p = e / z
    iota = lax.broadcasted_iota(jnp.int32, (rb, v), 1)
    amax = rmax(p)
    tok_ref[...] = rmin(jnp.where(p == amax, iota, v))
    topk = tk_ref[...]
    topp = tp_ref[...]

    def body(_, carry):
        lo, hi = carry
        mid = (lo + hi) >> 1
        midf = lax.bitcast_convert_type(mid, jnp.float32)
        cs, ss = [], []
        for s, e2 in bounds:
            pc = p[:, s:e2]
            mc = pc > midf
            cs.append(jnp.sum(mc.astype(jnp.int32), axis=1, keepdims=True))
            ss.append(jnp.sum(jnp.where(mc, pc, 0.0), axis=1, keepdims=True))
        cnt = functools.reduce(jnp.add, cs)
        sp = functools.reduce(jnp.add, ss)
        ok = (cnt < topk) & (sp <= topp)
        return jnp.where(ok, lo, mid), jnp.where(ok, mid, hi)

    lo0 = jnp.full((rb, 1), -1, jnp.int32)
    hi0 = jnp.full((rb, 1), _ONE_KEY, jnp.int32)
    _, hi = lax.fori_loop(0, 30, body, (lo0, hi0))
    kept = p >= lax.bitcast_convert_type(hi, jnp.float32)
    s = rsum(jnp.where(kept, p, 0.0))
    out_ref[...] = jnp.where(kept, p * (1.0 / s), 0.0)


def kernel(logits, output_tokens, presence_penalties, frequency_penalties,
           temperatures, top_ps, top_ks):
    b, v = logits.shape
    h = output_tokens.shape[1]
    rb = 8
    rbm = 16

    tok_pad, s1, s2, fst = pl.pallas_call(
        _prep_body,
        grid=(b // rb,),
        in_specs=[
            pl.BlockSpec((rb, h), lambda i: (i, 0)),
            pl.BlockSpec((rb, 1), lambda i: (i, 0)),
            pl.BlockSpec((rb, 1), lambda i: (i, 0)),
        ],
        out_specs=[pl.BlockSpec((rb, _PADH), lambda i: (i, 0))] * 4,
        out_shape=[
            jax.ShapeDtypeStruct((b, _PADH), jnp.int32),
            jax.ShapeDtypeStruct((b, _PADH), jnp.float32),
            jax.ShapeDtypeStruct((b, _PADH), jnp.float32),
            jax.ShapeDtypeStruct((b, _PADH), jnp.int32),
        ],
    )(output_tokens,
      frequency_penalties.reshape(b, 1),
      presence_penalties.reshape(b, 1))

    sc_pen = functools.partial(
        pl.kernel,
        mesh=plsc.VectorSubcoreMesh(core_axis_name="c", subcore_axis_name="s"),
        compiler_params=pltpu.CompilerParams(needs_layout_passes=False),
        out_type=jax.ShapeDtypeStruct((b, v), jnp.float32),
        scratch_types=[
            pltpu.VMEM((v,), jnp.float32),
            pltpu.VMEM((_PADH,), jnp.int32),
            pltpu.VMEM((_PADH,), jnp.float32),
            pltpu.VMEM((_PADH,), jnp.float32),
            pltpu.VMEM((_PADH,), jnp.int32),
        ],
    )(_sc_penalize_body)
    lpen = sc_pen(tok_pad, s1, s2, fst, logits)

    probs, tok = pl.pallas_call(
        _main_body,
        grid=(b // rbm,),
        in_specs=[
            pl.BlockSpec((rbm, v), lambda i: (i, 0)),
            pl.BlockSpec((rbm, 1), lambda i: (i, 0)),
            pl.BlockSpec((rbm, 1), lambda i: (i, 0)),
            pl.BlockSpec((rbm, 1), lambda i: (i, 0)),
        ],
        out_specs=[
            pl.BlockSpec((rbm, v), lambda i: (i, 0)),
            pl.BlockSpec((rbm, 1), lambda i: (i, 0)),
        ],
        out_shape=[
            jax.ShapeDtypeStruct((b, v), jnp.float32),
            jax.ShapeDtypeStruct((b, 1), jnp.int32),
        ],
    )(lpen,
      temperatures.reshape(b, 1),
      top_ps.reshape(b, 1),
      top_ks.reshape(b, 1))

    return probs, tok.reshape(b)
